# baseline (speedup 1.0000x reference)
import jax
import jax.numpy as jnp
from jax import lax
from jax.experimental import pallas as pl
from jax.experimental.pallas import tpu as pltpu
from jax.experimental.pallas import tpu_sc as plsc

_B, _N, _D, _E = 4, 10000, 128, 160000
_NP = _N + 16
_NT = 16
_NC = 2
_EPT = _E // _NT
_CH = 128
_NCHK = 79
_STG = 40
_K = _N // 2
_BLK = 2000
_NPAD = 10240

_f32 = jnp.float32
_i32 = jnp.int32


_sc_mesh = plsc.VectorSubcoreMesh(core_axis_name="c", subcore_axis_name="s")


def _agg_body(y, srcp, dstp, zeros, out, agg, idxs, idxd, rows, sem0, sem1):
    c = lax.axis_index("c")
    s = lax.axis_index("s")
    for i in range(_B // _NC):
        b = c * (_B // _NC) + i
        @pl.when(s < _NT - 1)
        def _():
            for z in range(5):
                pltpu.sync_copy(zeros,
                                agg.at[pl.ds(s * 640 + z * 128, 128)])

        @pl.when(s == _NT - 1)
        def _():
            for z in range(3):
                pltpu.sync_copy(zeros,
                                agg.at[pl.ds(9600 + z * 128, 128)])
            pltpu.sync_copy(zeros.at[pl.ds(0, 32)],
                            agg.at[pl.ds(9984, 32)])

        plsc.subcore_barrier()
        blk = b * _NT + s

        r0, r1 = rows.at[0], rows.at[1]
        for off, size in ((0, _STG), (_STG, _NCHK - _STG)):
            pltpu.sync_copy(srcp.at[blk, pl.ds(off, size)],
                            idxs.at[pl.ds(0, size)])
            pltpu.sync_copy(dstp.at[blk, pl.ds(off, size)],
                            idxd.at[pl.ds(0, size)])
            pltpu.async_copy(y.at[idxs.at[0]], r0, sem0)
            pltpu.async_copy(y.at[idxs.at[1]], r1, sem1)

            def pair(p, carry):
                j = 2 * p
                pltpu.make_async_copy(y.at[idxs.at[j]], r0, sem0).wait()
                pltpu.sync_copy(r0, agg.at[idxd.at[j]], add=True)
                pltpu.async_copy(y.at[idxs.at[j + 2]], r0, sem0)
                pltpu.make_async_copy(y.at[idxs.at[j + 1]], r1, sem1).wait()
                pltpu.sync_copy(r1, agg.at[idxd.at[j + 1]], add=True)
                pltpu.async_copy(y.at[idxs.at[j + 3]], r1, sem1)
                return carry

            if size % 2 == 0:
                lax.fori_loop(0, (size - 2) // 2, pair, 0)
                pltpu.make_async_copy(y.at[idxs.at[size - 2]], r0, sem0).wait()
                pltpu.sync_copy(r0, agg.at[idxd.at[size - 2]], add=True)
                pltpu.make_async_copy(y.at[idxs.at[size - 1]], r1, sem1).wait()
                pltpu.sync_copy(r1, agg.at[idxd.at[size - 1]], add=True)
            else:
                lax.fori_loop(0, (size - 3) // 2, pair, 0)
                pltpu.make_async_copy(y.at[idxs.at[size - 3]], r0, sem0).wait()
                pltpu.sync_copy(r0, agg.at[idxd.at[size - 3]], add=True)
                pltpu.async_copy(y.at[idxs.at[size - 1]], r0, sem0)
                pltpu.make_async_copy(y.at[idxs.at[size - 2]], r1, sem1).wait()
                pltpu.sync_copy(r1, agg.at[idxd.at[size - 2]], add=True)
                pltpu.make_async_copy(y.at[idxs.at[size - 1]], r0, sem0).wait()
                pltpu.sync_copy(r0, agg.at[idxd.at[size - 1]], add=True)
        plsc.subcore_barrier()
        @pl.when(s < _NT - 1)
        def _():
            pltpu.sync_copy(agg.at[pl.ds(s * 640, 640)],
                            out.at[b, pl.ds(s * 640, 640)])

        @pl.when(s == _NT - 1)
        def _():
            pltpu.sync_copy(agg.at[pl.ds(9600, 400)],
                            out.at[b, pl.ds(9600, 400)])

        plsc.subcore_barrier()


_agg_call = pl.kernel(
    _agg_body,
    out_type=jax.ShapeDtypeStruct((_B, _N, _D), _f32),
    mesh=_sc_mesh,
    scratch_types=[
        pltpu.VMEM_SHARED((_NP, _D), _f32),
        pltpu.VMEM((_STG, _CH), _i32),
        pltpu.VMEM((_STG, _CH), _i32),
        pltpu.VMEM((2, _CH, _D), _f32),
        pltpu.SemaphoreType.DMA,
        pltpu.SemaphoreType.DMA,
    ],
)


_NPC = 10240


def _cnt_body(dstp, zvec, ones_h, out, csh, idxd, onesv):
    c = lax.axis_index("c")
    s = lax.axis_index("s")
    pltpu.sync_copy(ones_h, onesv)
    for i in range(_B // _NC):
        b = c * (_B // _NC) + i
        pltpu.sync_copy(zvec.at[pl.ds(0, _NPC // _NT)],
                        csh.at[pl.ds(s * (_NPC // _NT), _NPC // _NT)])
        plsc.subcore_barrier()
        pltpu.sync_copy(dstp.at[b * _NT + s], idxd)

        def chunk(j, carry):
            pltpu.sync_copy(onesv, csh.at[idxd.at[j]], add=True)
            return carry

        lax.fori_loop(0, _NCHK, chunk, 0)
        plsc.subcore_barrier()
        pltpu.sync_copy(csh.at[pl.ds(s * 640, 640)],
                        out.at[b, pl.ds(s * 640, 640)])
        plsc.subcore_barrier()


_cnt_call = pl.kernel(
    _cnt_body,
    out_type=jax.ShapeDtypeStruct((_B, _NPC), _f32),
    mesh=_sc_mesh,
    scratch_types=[
        pltpu.VMEM_SHARED((_NPC,), _f32),
        pltpu.VMEM((_NCHK, _CH), _i32),
        pltpu.VMEM((_CH,), _f32),
    ],
)


def _t0_body(x_ref, wcat_ref, z_ref, w_ref):
    m = jnp.dot(x_ref[...], wcat_ref[...], preferred_element_type=_f32)
    z_ref[...] = m[:, :_D]
    w_ref[...] = m[:, _D:]


def _tc_first(x, wcat):
    grid = (_B * _N // _BLK,)
    return pl.pallas_call(
        _t0_body,
        grid=grid,
        in_specs=[
            pl.BlockSpec((_BLK, _D), lambda i: (i, 0)),
            pl.BlockSpec((_D, 2 * _D), lambda i: (0, 0)),
        ],
        out_specs=[
            pl.BlockSpec((_BLK, _D), lambda i: (i, 0)),
            pl.BlockSpec((_BLK, _D), lambda i: (i, 0)),
        ],
        out_shape=[
            jax.ShapeDtypeStruct((_B * _N, _D), _f32),
            jax.ShapeDtypeStruct((_B * _N, _D), _f32),
        ],
    )(x, wcat)


def _tmid_body(a_ref, wp_ref, cnt_ref, bl_ref, wcat_ref, z_ref, w_ref):
    inv = 1.0 / jnp.maximum(cnt_ref[...], 1.0)
    x = jnp.maximum(a_ref[...] * inv + wp_ref[...] + bl_ref[...], 0.0)
    m = jnp.dot(x, wcat_ref[...], preferred_element_type=_f32)
    z_ref[...] = m[:, :_D]
    w_ref[...] = m[:, _D:]


def _tc_mid(a, wp, cnt2, bl, wcat):
    grid = (_B * _N // _BLK,)
    return pl.pallas_call(
        _tmid_body,
        grid=grid,
        in_specs=[
            pl.BlockSpec((_BLK, _D), lambda i: (i, 0)),
            pl.BlockSpec((_BLK, _D), lambda i: (i, 0)),
            pl.BlockSpec((_BLK, 1), lambda i: (i, 0)),
            pl.BlockSpec((1, _D), lambda i: (0, 0)),
            pl.BlockSpec((_D, 2 * _D), lambda i: (0, 0)),
        ],
        out_specs=[
            pl.BlockSpec((_BLK, _D), lambda i: (i, 0)),
            pl.BlockSpec((_BLK, _D), lambda i: (i, 0)),
        ],
        out_shape=[
            jax.ShapeDtypeStruct((_B * _N, _D), _f32),
            jax.ShapeDtypeStruct((_B * _N, _D), _f32),
        ],
    )(a, wp, cnt2, bl, wcat)


def _f1_body(a_ref, wp_ref, cnt_ref, bl_ref, pw_ref, x3_ref, sc_ref):
    inv = 1.0 / jnp.maximum(cnt_ref[...], 1.0)
    x3 = a_ref[...] * inv + wp_ref[...] + bl_ref[...]
    sc_ref[...] = jnp.dot(x3, pw_ref[...], preferred_element_type=_f32)
    x3_ref[...] = x3


def _tc_f1(a, wp, cnt2, bl, pw_row):
    grid = (_B * _N // _BLK,)
    return pl.pallas_call(
        _f1_body,
        grid=grid,
        in_specs=[
            pl.BlockSpec((_BLK, _D), lambda i: (i, 0)),
            pl.BlockSpec((_BLK, _D), lambda i: (i, 0)),
            pl.BlockSpec((_BLK, 1), lambda i: (i, 0)),
            pl.BlockSpec((1, _D), lambda i: (0, 0)),
            pl.BlockSpec((_D, 1), lambda i: (0, 0)),
        ],
        out_specs=[
            pl.BlockSpec((_BLK, _D), lambda i: (i, 0)),
            pl.BlockSpec((_BLK, 1), lambda i: (i, 0)),
        ],
        out_shape=[
            jax.ShapeDtypeStruct((_B * _N, _D), _f32),
            jax.ShapeDtypeStruct((_B * _N, 1), _f32),
        ],
    )(a, wp, cnt2, bl, pw_row)


def _f3_body(x3_ref, sc_ref, pw_ref, w1_ref, b1_ref, w2_ref, b2_ref, out_ref,
             pooled):
    b = pl.program_id(0)
    sc = sc_ref[0]
    bits = lax.bitcast_convert_type(sc, _i32)
    ukey = jnp.where(bits >= 0, bits, jnp.int32(-2147483648) - bits)

    def srch(_, lohi):
        lo, hi = lohi
        d = hi - lo
        half = lax.shift_right_logical(d, 1) + (d & 1)
        mid = lo + half
        cge = jnp.sum((ukey >= mid).astype(_i32))
        ok = cge >= _K
        return jnp.where(ok, mid, lo), jnp.where(ok, hi, mid - 1)

    t_key, _hi = lax.fori_loop(0, 32, srch, (jnp.int32(-2147483648),
                                             jnp.int32(2147483647)))
    cgt = jnp.sum((ukey > t_key).astype(_i32))
    r = _K - cgt
    pos = lax.broadcasted_iota(_i32, (1, _NPAD), 1)
    iseq = ukey == t_key

    def jsr(_, lohi):
        lo, hi = lohi
        mid = (lo + hi) // 2
        ce = jnp.sum((iseq & (pos <= mid)).astype(_i32))
        ok = ce >= r
        return jnp.where(ok, lo, mid + 1), jnp.where(ok, mid, hi)

    j_pos, _jhi = lax.fori_loop(0, 14, jsr, (jnp.int32(0),
                                             jnp.int32(_NPAD - 1)))
    mask = (ukey > t_key) | (iseq & (pos <= j_pos))
    pw = pw_ref[...]
    nrm = jnp.sqrt(jnp.sum(pw * pw))
    vals = jnp.tanh(sc / nrm)
    coeff = jnp.where(mask, vals, 0.0) * (1.0 / _K)
    partial = jnp.dot(coeff, x3_ref[0], preferred_element_type=_f32)
    rowi = lax.broadcasted_iota(_i32, (8, _D), 0)
    pooled[...] = jnp.where(rowi == b, jnp.broadcast_to(partial, (8, _D)),
                            pooled[...])

    @pl.when(b == _B - 1)
    def _():
        h = jnp.maximum(
            jnp.dot(pooled[...], w1_ref[...], preferred_element_type=_f32)
            + b1_ref[...], 0.0)
        out_ref[...] = (jnp.dot(h, w2_ref[...], preferred_element_type=_f32)
                        + b2_ref[...])


def _tc_f3(x3p, scp, pw_row, w1, b1row, w2p, b2row):
    mh = w1.shape[1]
    return pl.pallas_call(
        _f3_body,
        grid=(_B,),
        in_specs=[
            pl.BlockSpec((1, _NPAD, _D), lambda b: (b, 0, 0)),
            pl.BlockSpec((1, 1, _NPAD), lambda b: (b, 0, 0)),
            pl.BlockSpec((1, _D), lambda b: (0, 0)),
            pl.BlockSpec((_D, mh), lambda b: (0, 0)),
            pl.BlockSpec((1, mh), lambda b: (0, 0)),
            pl.BlockSpec((mh, _D), lambda b: (0, 0)),
            pl.BlockSpec((1, _D), lambda b: (0, 0)),
        ],
        out_specs=pl.BlockSpec((8, _D), lambda b: (0, 0)),
        out_shape=jax.ShapeDtypeStruct((8, _D), _f32),
        scratch_shapes=[pltpu.VMEM((8, _D), _f32)],
    )(x3p, scp, pw_row, w1, b1row, w2p, b2row)


def kernel(freq, edge_index, edge_weight, Wl0, bl0, Wr0, Wl1, bl1, Wr1,
           Wl2, bl2, Wr2, pool_w, W1, b1, W2, b2):
    del edge_weight
    x0 = freq.reshape(_B * _N, _D)

    epad = _NCHK * _CH - _EPT
    offs = (jnp.arange(_B, dtype=_i32) * _N)[:, None]
    src_g = (edge_index[:, 0, :].astype(_i32) + offs).reshape(_B, _NT, _EPT)
    dst_l = edge_index[:, 1, :].astype(_i32).reshape(_B, _NT, _EPT)
    src_pad = jnp.pad(src_g, ((0, 0), (0, 0), (0, epad))) \
        .reshape(_B * _NT, _NCHK, _CH)
    padrows = _N + (jnp.arange(epad, dtype=_i32) % 16)
    padrows = jnp.broadcast_to(padrows, (_B, _NT, epad))
    dst_pad = jnp.concatenate([dst_l, padrows], axis=2) \
        .reshape(_B * _NT, _NCHK, _CH)
    zeros128 = jnp.zeros((128, _D), _f32)

    zvec = jnp.zeros((_NPC // _NT,), _f32)
    ones_h = jnp.ones((_CH,), _f32)
    cnt = _cnt_call(dst_pad, zvec, ones_h)[:, :_N]
    cnt2 = cnt.reshape(_B * _N, 1)

    wcat0 = jnp.concatenate([Wl0, Wr0], axis=1)
    wcat1 = jnp.concatenate([Wl1, Wr1], axis=1)
    wcat2 = jnp.concatenate([Wl2, Wr2], axis=1)

    z0, w0 = _tc_first(x0, wcat0)
    a0 = _agg_call(z0, src_pad, dst_pad, zeros128).reshape(_B * _N, _D)
    z1, w1 = _tc_mid(a0, w0, cnt2, bl0.reshape(1, _D), wcat1)
    a1 = _agg_call(z1, src_pad, dst_pad, zeros128).reshape(_B * _N, _D)
    z2, w2 = _tc_mid(a1, w1, cnt2, bl1.reshape(1, _D), wcat2)
    a2 = _agg_call(z2, src_pad, dst_pad, zeros128).reshape(_B * _N, _D)

    x3, score = _tc_f1(a2, w2, cnt2, bl2.reshape(1, _D),
                       pool_w.reshape(_D, 1))

    x3p = jnp.pad(x3.reshape(_B, _N, _D),
                  ((0, 0), (0, _NPAD - _N), (0, 0)))
    scp = jnp.pad(score.reshape(_B, _N), ((0, 0), (0, _NPAD - _N)),
                  constant_values=-jnp.inf).reshape(_B, 1, _NPAD)

    w2p = jnp.pad(W2, ((0, 0), (0, _D - W2.shape[1])))
    b2p = jnp.pad(b2, ((0, _D - b2.shape[0]))).reshape(1, _D)
    outp = _tc_f3(x3p, scp, pool_w.reshape(1, _D), W1,
                  b1.reshape(1, W1.shape[1]), w2p, b2p)
    return outp[:_B, :b2.shape[0]]


# speedup vs baseline: 1.0009x; 1.0009x over previous
import jax
import jax.numpy as jnp
from jax import lax
from jax.experimental import pallas as pl
from jax.experimental.pallas import tpu as pltpu
from jax.experimental.pallas import tpu_sc as plsc

_B, _N, _D, _E = 4, 10000, 128, 160000
_NP = _N + 16
_NT = 16
_NC = 2
_EPT = _E // _NT
_CH = 128
_NCHK = 79
_STG = 40
_K = _N // 2
_BLK = 2000
_NPAD = 10240

_f32 = jnp.float32
_i32 = jnp.int32


def _sc_mesh():
    return plsc.VectorSubcoreMesh(core_axis_name="c", subcore_axis_name="s")


def _agg_body(y, srcp, dstp, zeros, out, agg, idxs, idxd, rows, sem0, sem1):
    c = lax.axis_index("c")
    s = lax.axis_index("s")
    for i in range(_B // _NC):
        b = c * (_B // _NC) + i
        @pl.when(s < _NT - 1)
        def _():
            for z in range(5):
                pltpu.sync_copy(zeros,
                                agg.at[pl.ds(s * 640 + z * 128, 128)])

        @pl.when(s == _NT - 1)
        def _():
            for z in range(3):
                pltpu.sync_copy(zeros,
                                agg.at[pl.ds(9600 + z * 128, 128)])
            pltpu.sync_copy(zeros.at[pl.ds(0, 32)],
                            agg.at[pl.ds(9984, 32)])

        plsc.subcore_barrier()
        blk = b * _NT + s

        r0, r1 = rows.at[0], rows.at[1]
        for off, size in ((0, _STG), (_STG, _NCHK - _STG)):
            pltpu.sync_copy(srcp.at[blk, pl.ds(off, size)],
                            idxs.at[pl.ds(0, size)])
            pltpu.sync_copy(dstp.at[blk, pl.ds(off, size)],
                            idxd.at[pl.ds(0, size)])
            pltpu.async_copy(y.at[idxs.at[0]], r0, sem0)
            pltpu.async_copy(y.at[idxs.at[1]], r1, sem1)

            def pair(p, carry):
                j = 2 * p
                pltpu.make_async_copy(y.at[idxs.at[j]], r0, sem0).wait()
                pltpu.sync_copy(r0, agg.at[idxd.at[j]], add=True)
                pltpu.async_copy(y.at[idxs.at[j + 2]], r0, sem0)
                pltpu.make_async_copy(y.at[idxs.at[j + 1]], r1, sem1).wait()
                pltpu.sync_copy(r1, agg.at[idxd.at[j + 1]], add=True)
                pltpu.async_copy(y.at[idxs.at[j + 3]], r1, sem1)
                return carry

            if size % 2 == 0:
                lax.fori_loop(0, (size - 2) // 2, pair, 0)
                pltpu.make_async_copy(y.at[idxs.at[size - 2]], r0, sem0).wait()
                pltpu.sync_copy(r0, agg.at[idxd.at[size - 2]], add=True)
                pltpu.make_async_copy(y.at[idxs.at[size - 1]], r1, sem1).wait()
                pltpu.sync_copy(r1, agg.at[idxd.at[size - 1]], add=True)
            else:
                lax.fori_loop(0, (size - 3) // 2, pair, 0)
                pltpu.make_async_copy(y.at[idxs.at[size - 3]], r0, sem0).wait()
                pltpu.sync_copy(r0, agg.at[idxd.at[size - 3]], add=True)
                pltpu.async_copy(y.at[idxs.at[size - 1]], r0, sem0)
                pltpu.make_async_copy(y.at[idxs.at[size - 2]], r1, sem1).wait()
                pltpu.sync_copy(r1, agg.at[idxd.at[size - 2]], add=True)
                pltpu.make_async_copy(y.at[idxs.at[size - 1]], r0, sem0).wait()
                pltpu.sync_copy(r0, agg.at[idxd.at[size - 1]], add=True)
        plsc.subcore_barrier()
        @pl.when(s < _NT - 1)
        def _():
            pltpu.sync_copy(agg.at[pl.ds(s * 640, 640)],
                            out.at[b, pl.ds(s * 640, 640)])

        @pl.when(s == _NT - 1)
        def _():
            pltpu.sync_copy(agg.at[pl.ds(9600, 400)],
                            out.at[b, pl.ds(9600, 400)])

        plsc.subcore_barrier()


def _agg_call(*args):
    return pl.kernel(
        _agg_body,
        out_type=jax.ShapeDtypeStruct((_B, _N, _D), _f32),
        mesh=_sc_mesh(),
        scratch_types=[
            pltpu.VMEM_SHARED((_NP, _D), _f32),
            pltpu.VMEM((_STG, _CH), _i32),
            pltpu.VMEM((_STG, _CH), _i32),
            pltpu.VMEM((2, _CH, _D), _f32),
            pltpu.SemaphoreType.DMA,
            pltpu.SemaphoreType.DMA,
        ],
    )(*args)


_NPC = 10240


def _cnt_body(dstp, zvec, ones_h, out, csh, idxd, onesv):
    c = lax.axis_index("c")
    s = lax.axis_index("s")
    pltpu.sync_copy(ones_h, onesv)
    for i in range(_B // _NC):
        b = c * (_B // _NC) + i
        pltpu.sync_copy(zvec.at[pl.ds(0, _NPC // _NT)],
                        csh.at[pl.ds(s * (_NPC // _NT), _NPC // _NT)])
        plsc.subcore_barrier()
        pltpu.sync_copy(dstp.at[b * _NT + s], idxd)

        def chunk(j, carry):
            pltpu.sync_copy(onesv, csh.at[idxd.at[j]], add=True)
            return carry

        lax.fori_loop(0, _NCHK, chunk, 0)
        plsc.subcore_barrier()
        pltpu.sync_copy(csh.at[pl.ds(s * 640, 640)],
                        out.at[b, pl.ds(s * 640, 640)])
        plsc.subcore_barrier()


def _cnt_call(*args):
    return pl.kernel(
        _cnt_body,
        out_type=jax.ShapeDtypeStruct((_B, _NPC), _f32),
        mesh=_sc_mesh(),
        scratch_types=[
            pltpu.VMEM_SHARED((_NPC,), _f32),
            pltpu.VMEM((_NCHK, _CH), _i32),
            pltpu.VMEM((_CH,), _f32),
        ],
    )(*args)


def _t0_body(x_ref, wcat_ref, z_ref, w_ref):
    m = jnp.dot(x_ref[...], wcat_ref[...], preferred_element_type=_f32)
    z_ref[...] = m[:, :_D]
    w_ref[...] = m[:, _D:]


def _tc_first(x, wcat):
    grid = (_B * _N // _BLK,)
    return pl.pallas_call(
        _t0_body,
        grid=grid,
        in_specs=[
            pl.BlockSpec((_BLK, _D), lambda i: (i, 0)),
            pl.BlockSpec((_D, 2 * _D), lambda i: (0, 0)),
        ],
        out_specs=[
            pl.BlockSpec((_BLK, _D), lambda i: (i, 0)),
            pl.BlockSpec((_BLK, _D), lambda i: (i, 0)),
        ],
        out_shape=[
            jax.ShapeDtypeStruct((_B * _N, _D), _f32),
            jax.ShapeDtypeStruct((_B * _N, _D), _f32),
        ],
    )(x, wcat)


def _tmid_body(a_ref, wp_ref, cnt_ref, bl_ref, wcat_ref, z_ref, w_ref):
    inv = 1.0 / jnp.maximum(cnt_ref[...], 1.0)
    x = jnp.maximum(a_ref[...] * inv + wp_ref[...] + bl_ref[...], 0.0)
    m = jnp.dot(x, wcat_ref[...], preferred_element_type=_f32)
    z_ref[...] = m[:, :_D]
    w_ref[...] = m[:, _D:]


def _tc_mid(a, wp, cnt2, bl, wcat):
    grid = (_B * _N // _BLK,)
    return pl.pallas_call(
        _tmid_body,
        grid=grid,
        in_specs=[
            pl.BlockSpec((_BLK, _D), lambda i: (i, 0)),
            pl.BlockSpec((_BLK, _D), lambda i: (i, 0)),
            pl.BlockSpec((_BLK, 1), lambda i: (i, 0)),
            pl.BlockSpec((1, _D), lambda i: (0, 0)),
            pl.BlockSpec((_D, 2 * _D), lambda i: (0, 0)),
        ],
        out_specs=[
            pl.BlockSpec((_BLK, _D), lambda i: (i, 0)),
            pl.BlockSpec((_BLK, _D), lambda i: (i, 0)),
        ],
        out_shape=[
            jax.ShapeDtypeStruct((_B * _N, _D), _f32),
            jax.ShapeDtypeStruct((_B * _N, _D), _f32),
        ],
    )(a, wp, cnt2, bl, wcat)


def _f1_body(a_ref, wp_ref, cnt_ref, bl_ref, pw_ref, x3_ref, sc_ref):
    inv = 1.0 / jnp.maximum(cnt_ref[...], 1.0)
    x3 = a_ref[...] * inv + wp_ref[...] + bl_ref[...]
    sc_ref[...] = jnp.dot(x3, pw_ref[...], preferred_element_type=_f32)
    x3_ref[...] = x3


def _tc_f1(a, wp, cnt2, bl, pw_row):
    grid = (_B * _N // _BLK,)
    return pl.pallas_call(
        _f1_body,
        grid=grid,
        in_specs=[
            pl.BlockSpec((_BLK, _D), lambda i: (i, 0)),
            pl.BlockSpec((_BLK, _D), lambda i: (i, 0)),
            pl.BlockSpec((_BLK, 1), lambda i: (i, 0)),
            pl.BlockSpec((1, _D), lambda i: (0, 0)),
            pl.BlockSpec((_D, 1), lambda i: (0, 0)),
        ],
        out_specs=[
            pl.BlockSpec((_BLK, _D), lambda i: (i, 0)),
            pl.BlockSpec((_BLK, 1), lambda i: (i, 0)),
        ],
        out_shape=[
            jax.ShapeDtypeStruct((_B * _N, _D), _f32),
            jax.ShapeDtypeStruct((_B * _N, 1), _f32),
        ],
    )(a, wp, cnt2, bl, pw_row)


def _f3_body(x3_ref, sc_ref, pw_ref, w1_ref, b1_ref, w2_ref, b2_ref, out_ref,
             pooled):
    b = pl.program_id(0)
    sc = sc_ref[0]
    bits = lax.bitcast_convert_type(sc, _i32)
    ukey = jnp.where(bits >= 0, bits, jnp.int32(-2147483648) - bits)

    def srch(_, lohi):
        lo, hi = lohi
        d = hi - lo
        half = lax.shift_right_logical(d, 1) + (d & 1)
        mid = lo + half
        cge = jnp.sum((ukey >= mid).astype(_i32))
        ok = cge >= _K
        return jnp.where(ok, mid, lo), jnp.where(ok, hi, mid - 1)

    t_key, _hi = lax.fori_loop(0, 32, srch, (jnp.int32(-2147483648),
                                             jnp.int32(2147483647)))
    cgt = jnp.sum((ukey > t_key).astype(_i32))
    r = _K - cgt
    pos = lax.broadcasted_iota(_i32, (1, _NPAD), 1)
    iseq = ukey == t_key

    def jsr(_, lohi):
        lo, hi = lohi
        mid = (lo + hi) // 2
        ce = jnp.sum((iseq & (pos <= mid)).astype(_i32))
        ok = ce >= r
        return jnp.where(ok, lo, mid + 1), jnp.where(ok, mid, hi)

    j_pos, _jhi = lax.fori_loop(0, 14, jsr, (jnp.int32(0),
                                             jnp.int32(_NPAD - 1)))
    mask = (ukey > t_key) | (iseq & (pos <= j_pos))
    pw = pw_ref[...]
    nrm = jnp.sqrt(jnp.sum(pw * pw))
    vals = jnp.tanh(sc / nrm)
    coeff = jnp.where(mask, vals, 0.0) * (1.0 / _K)
    partial = jnp.dot(coeff, x3_ref[0], preferred_element_type=_f32)
    rowi = lax.broadcasted_iota(_i32, (8, _D), 0)
    pooled[...] = jnp.where(rowi == b, jnp.broadcast_to(partial, (8, _D)),
                            pooled[...])

    @pl.when(b == _B - 1)
    def _():
        h = jnp.maximum(
            jnp.dot(pooled[...], w1_ref[...], preferred_element_type=_f32)
            + b1_ref[...], 0.0)
        out_ref[...] = (jnp.dot(h, w2_ref[...], preferred_element_type=_f32)
                        + b2_ref[...])


def _tc_f3(x3p, scp, pw_row, w1, b1row, w2p, b2row):
    mh = w1.shape[1]
    return pl.pallas_call(
        _f3_body,
        grid=(_B,),
        in_specs=[
            pl.BlockSpec((1, _NPAD, _D), lambda b: (b, 0, 0)),
            pl.BlockSpec((1, 1, _NPAD), lambda b: (b, 0, 0)),
            pl.BlockSpec((1, _D), lambda b: (0, 0)),
            pl.BlockSpec((_D, mh), lambda b: (0, 0)),
            pl.BlockSpec((1, mh), lambda b: (0, 0)),
            pl.BlockSpec((mh, _D), lambda b: (0, 0)),
            pl.BlockSpec((1, _D), lambda b: (0, 0)),
        ],
        out_specs=pl.BlockSpec((8, _D), lambda b: (0, 0)),
        out_shape=jax.ShapeDtypeStruct((8, _D), _f32),
        scratch_shapes=[pltpu.VMEM((8, _D), _f32)],
    )(x3p, scp, pw_row, w1, b1row, w2p, b2row)


def kernel(freq, edge_index, edge_weight, Wl0, bl0, Wr0, Wl1, bl1, Wr1,
           Wl2, bl2, Wr2, pool_w, W1, b1, W2, b2):
    del edge_weight
    x0 = freq.reshape(_B * _N, _D)

    epad = _NCHK * _CH - _EPT
    offs = (jnp.arange(_B, dtype=_i32) * _N)[:, None]
    src_g = (edge_index[:, 0, :].astype(_i32) + offs).reshape(_B, _NT, _EPT)
    dst_l = edge_index[:, 1, :].astype(_i32).reshape(_B, _NT, _EPT)
    src_pad = jnp.pad(src_g, ((0, 0), (0, 0), (0, epad))) \
        .reshape(_B * _NT, _NCHK, _CH)
    padrows = _N + (jnp.arange(epad, dtype=_i32) % 16)
    padrows = jnp.broadcast_to(padrows, (_B, _NT, epad))
    dst_pad = jnp.concatenate([dst_l, padrows], axis=2) \
        .reshape(_B * _NT, _NCHK, _CH)
    zeros128 = jnp.zeros((128, _D), _f32)

    zvec = jnp.zeros((_NPC // _NT,), _f32)
    ones_h = jnp.ones((_CH,), _f32)
    cnt = _cnt_call(dst_pad, zvec, ones_h)[:, :_N]
    cnt2 = cnt.reshape(_B * _N, 1)

    wcat0 = jnp.concatenate([Wl0, Wr0], axis=1)
    wcat1 = jnp.concatenate([Wl1, Wr1], axis=1)
    wcat2 = jnp.concatenate([Wl2, Wr2], axis=1)

    z0, w0 = _tc_first(x0, wcat0)
    a0 = _agg_call(z0, src_pad, dst_pad, zeros128).reshape(_B * _N, _D)
    z1, w1 = _tc_mid(a0, w0, cnt2, bl0.reshape(1, _D), wcat1)
    a1 = _agg_call(z1, src_pad, dst_pad, zeros128).reshape(_B * _N, _D)
    z2, w2 = _tc_mid(a1, w1, cnt2, bl1.reshape(1, _D), wcat2)
    a2 = _agg_call(z2, src_pad, dst_pad, zeros128).reshape(_B * _N, _D)

    x3, score = _tc_f1(a2, w2, cnt2, bl2.reshape(1, _D),
                       pool_w.reshape(_D, 1))

    x3p = jnp.pad(x3.reshape(_B, _N, _D),
                  ((0, 0), (0, _NPAD - _N), (0, 0)))
    scp = jnp.pad(score.reshape(_B, _N), ((0, 0), (0, _NPAD - _N)),
                  constant_values=-jnp.inf).reshape(_B, 1, _NPAD)

    w2p = jnp.pad(W2, ((0, 0), (0, _D - W2.shape[1])))
    b2p = jnp.pad(b2, ((0, _D - b2.shape[0]))).reshape(1, _D)
    outp = _tc_f3(x3p, scp, pool_w.reshape(1, _D), W1,
                  b1.reshape(1, W1.shape[1]), w2p, b2p)
    return outp[:_B, :b2.shape[0]]
